# Initial kernel scaffold; baseline (speedup 1.0000x reference)
import jax
import jax.numpy as jnp
from jax.experimental import pallas as pl


def kernel(inputs, edge_index, edge_weight, weight):
    raise NotImplementedError("write your pallas kernel here")


# speedup vs baseline: 4.3775x; 4.3775x over previous
import functools

import jax
import jax.numpy as jnp
from jax import lax
from jax.experimental import pallas as pl
from jax.experimental.pallas import tpu as pltpu
from jax.experimental.pallas import tpu_sc as plsc

N = 10000
E = 320000
D = 128

NC = 2
NS = 16
NW = NC * NS
EPT = E // NW
C = 80
NCHUNK = EPT // C
N_PAD = 10240
ROWS_PER_TILE = N_PAD // NS


def _matmul_body(x_ref, w_ref, o_ref):
    o_ref[...] = jnp.dot(x_ref[...], w_ref[...],
                         preferred_element_type=jnp.float32)


def _matmul(x, w):
    bm = 2000
    return pl.pallas_call(
        _matmul_body,
        grid=(N // bm,),
        in_specs=[pl.BlockSpec((bm, D), lambda i: (i, 0)),
                  pl.BlockSpec((D, D), lambda i: (0, 0))],
        out_specs=pl.BlockSpec((bm, D), lambda i: (i, 0)),
        out_shape=jax.ShapeDtypeStruct((N, D), jnp.float32),
    )(x, w)


def _add_body(a_ref, b_ref, o_ref):
    o_ref[...] = a_ref[...] + b_ref[...]


def _combine(partials):
    bm = 2000
    return pl.pallas_call(
        _add_body,
        grid=(N // bm,),
        in_specs=[pl.BlockSpec((bm, D), lambda i: (i, 0)),
                  pl.BlockSpec((bm, D), lambda i: (i, 0))],
        out_specs=pl.BlockSpec((bm, D), lambda i: (i, 0)),
        out_shape=jax.ShapeDtypeStruct((N, D), jnp.float32),
    )(partials[0], partials[1])


def _spmm_body(support_hbm, src_hbm, dst_hbm, ew_hbm, zeros_hbm, out_hbm,
               idx_v, dst_v, w_v, rows_v, acc_sh, sem):
    c = lax.axis_index("c")
    s = lax.axis_index("s")
    wid = c * NS + s

    row0 = s * ROWS_PER_TILE
    pltpu.sync_copy(zeros_hbm.at[pl.ds(row0, ROWS_PER_TILE)],
                    acc_sh.at[pl.ds(row0, ROWS_PER_TILE)])
    plsc.subcore_barrier()

    base0 = wid * EPT

    def chunk_body(k, carry):
        base = base0 + k * C
        pltpu.sync_copy(src_hbm.at[pl.ds(base, C)], idx_v)
        pltpu.sync_copy(dst_hbm.at[pl.ds(base, C)], dst_v)
        pltpu.sync_copy(ew_hbm.at[pl.ds(base, C)], w_v)
        pltpu.async_copy(support_hbm.at[idx_v], rows_v, sem).wait()

        def scale_body(g, carry2):
            e0 = g * 16
            wvec = w_v[pl.ds(e0, 16)]
            for l in range(16):
                wv = jnp.full((16,), wvec[l], jnp.float32)
                for j in range(D // 16):
                    sl = pl.ds(j * 16, 16)
                    rows_v[e0 + l, sl] = rows_v[e0 + l, sl] * wv
            return carry2

        lax.fori_loop(0, C // 16, scale_body, 0)
        pltpu.async_copy(rows_v, acc_sh.at[dst_v], sem, add=True).wait()
        return carry

    lax.fori_loop(0, NCHUNK, chunk_body, 0)

    plsc.subcore_barrier()
    pltpu.sync_copy(acc_sh.at[pl.ds(row0, ROWS_PER_TILE)],
                    out_hbm.at[c, pl.ds(row0, ROWS_PER_TILE)])


def _spmm(support, src, dst, ew, zeros):
    mesh = plsc.VectorSubcoreMesh(core_axis_name="c", subcore_axis_name="s")
    k = pl.kernel(
        _spmm_body,
        out_type=jax.ShapeDtypeStruct((NC, N_PAD, D), jnp.float32),
        mesh=mesh,
        scratch_types=[
            pltpu.VMEM((C,), jnp.int32),
            pltpu.VMEM((C,), jnp.int32),
            pltpu.VMEM((C,), jnp.float32),
            pltpu.VMEM((C, D), jnp.float32),
            pltpu.VMEM_SHARED((N_PAD, D), jnp.float32),
            pltpu.SemaphoreType.DMA,
        ],
    )
    return k(support, src, dst, ew, zeros)


def kernel(inputs, edge_index, edge_weight, weight):
    support = _matmul(inputs, weight)
    dst = edge_index[0]
    src = edge_index[1]
    zeros = jnp.zeros((N_PAD, D), jnp.float32)
    partials = _spmm(support, src, dst, edge_weight, zeros)
    return _combine(partials)


# speedup vs baseline: 11.7658x; 2.6878x over previous
import functools

import jax
import jax.numpy as jnp
from jax import lax
from jax.experimental import pallas as pl
from jax.experimental.pallas import tpu as pltpu
from jax.experimental.pallas import tpu_sc as plsc

N = 10000
E = 320000
D = 128

NC = 2
NS = 16
NW = NC * NS
EPT = E // NW
C = 80
NCHUNK = EPT // C
N_PAD = 10240
ROWS_PER_TILE = N_PAD // NS


def _matmul_body(x_ref, w_ref, o_ref):
    o_ref[...] = jnp.dot(x_ref[...], w_ref[...],
                         preferred_element_type=jnp.float32)


def _matmul(x, w):
    bm = 2000
    return pl.pallas_call(
        _matmul_body,
        grid=(N // bm,),
        in_specs=[pl.BlockSpec((bm, D), lambda i: (i, 0)),
                  pl.BlockSpec((D, D), lambda i: (0, 0))],
        out_specs=pl.BlockSpec((bm, D), lambda i: (i, 0)),
        out_shape=jax.ShapeDtypeStruct((N, D), jnp.float32),
    )(x, w)


def _add_body(a_ref, b_ref, o_ref):
    o_ref[...] = a_ref[...] + b_ref[...]


def _combine(partials):
    bm = 2000
    return pl.pallas_call(
        _add_body,
        grid=(N // bm,),
        in_specs=[pl.BlockSpec((bm, D), lambda i: (i, 0)),
                  pl.BlockSpec((bm, D), lambda i: (i, 0))],
        out_specs=pl.BlockSpec((bm, D), lambda i: (i, 0)),
        out_shape=jax.ShapeDtypeStruct((N, D), jnp.float32),
    )(partials[0], partials[1])


NBUF = 3
NEV = 6
NCHUNK_P = NCHUNK + 1
NBLK = NCHUNK_P // NEV


def _spmm_body(support_hbm, src_hbm, dst_hbm, ew_hbm, out_hbm,
               r0, r1, r2, e0_, e1_, e2_, e3_, e4_, e5_,
               d0, d1, d2, d3, d4, d5,
               w0, w1, w2, w3, w4, w5, acc_sh,
               g0, g1, g2, s0, s1, s2, x0, x1, x2, x3, x4, x5,
               y0, y1, y2, y3, y4, y5, z0, z1, z2, z3, z4, z5):
    c = lax.axis_index("c")
    sid = lax.axis_index("s")
    wid = c * NS + sid

    rows = [r0, r1, r2]
    srcv = [e0_, e1_, e2_, e3_, e4_, e5_]
    dstv = [d0, d1, d2, d3, d4, d5]
    zsem = [z0, z1, z2, z3, z4, z5]
    gsem = [g0, g1, g2]
    ssem = [s0, s1, s2]
    xsem = [x0, x1, x2, x3, x4, x5]
    wv = [w0, w1, w2, w3, w4, w5]
    ysem = [y0, y1, y2, y3, y4, y5]

    row0 = sid * ROWS_PER_TILE
    zv = jnp.zeros((16,), jnp.float32)

    def zrow(r, carry):
        for j in range(D // 16):
            r0[r, pl.ds(j * 16, 16)] = zv
        return carry

    lax.fori_loop(0, C, zrow, 0)
    for t in range(ROWS_PER_TILE // C):
        pltpu.sync_copy(r0, acc_sh.at[pl.ds(row0 + t * C, C)])
    plsc.subcore_barrier()

    def ebase(i):
        return wid * EPT + jnp.minimum(i, NCHUNK - 1) * C

    def issue_ev(i, b6):
        base = ebase(i)
        pltpu.async_copy(src_hbm.at[pl.ds(base, C)], srcv[b6], xsem[b6])
        pltpu.async_copy(dst_hbm.at[pl.ds(base, C)], dstv[b6], zsem[b6])
        pltpu.async_copy(ew_hbm.at[pl.ds(base, C)], wv[b6], ysem[b6])

    def wait_ev(b6):
        pltpu.make_async_copy(src_hbm.at[pl.ds(0, C)], srcv[b6],
                              xsem[b6]).wait()
        pltpu.make_async_copy(dst_hbm.at[pl.ds(0, C)], dstv[b6],
                              zsem[b6]).wait()
        pltpu.make_async_copy(ew_hbm.at[pl.ds(0, C)], wv[b6], ysem[b6]).wait()

    def issue_gather(b3, b6):
        pltpu.async_copy(support_hbm.at[srcv[b6]], rows[b3], gsem[b3])

    def wait_gather(b3):
        pltpu.make_async_copy(support_hbm.at[srcv[0]], rows[b3],
                              gsem[b3]).wait()

    def issue_scatter(b3, b6):
        pltpu.async_copy(rows[b3], acc_sh.at[dstv[b6]], ssem[b3], add=True)

    def wait_scatter(b3):
        pltpu.make_async_copy(support_hbm.at[srcv[0]], rows[b3],
                              ssem[b3]).wait()

    def scale(i, b3, b6):
        rb = rows[b3]
        wvb = wv[b6]
        valid = i < NCHUNK

        def group(g, carry):
            ge = g * 16
            wvec = jnp.where(valid, wvb[pl.ds(ge, 16)], zv)
            for l in range(16):
                wvl = jnp.full((16,), wvec[l], jnp.float32)
                for j in range(D // 16):
                    sl = pl.ds(j * 16, 16)
                    rb[ge + l, sl] = rb[ge + l, sl] * wvl
            return carry

        lax.fori_loop(0, C // 16, group, 0)

    def step(i, b3, b6):
        @pl.when(i >= 2)
        def _():
            wait_scatter((b3 + 1) % NBUF)

        @pl.when(i + 4 < NCHUNK_P)
        def _():
            issue_ev(i + 4, (b6 + 4) % NEV)

        @pl.when(i + 1 < NCHUNK_P)
        def _():
            wait_ev((b6 + 1) % NEV)
            issue_gather((b3 + 1) % NBUF, (b6 + 1) % NEV)

        wait_gather(b3)
        scale(i, b3, b6)
        issue_scatter(b3, b6)

    for j in range(4):
        issue_ev(j, j)
    wait_ev(0)
    issue_gather(0, 0)

    def block(q, carry):
        i0 = q * NEV
        for bb in range(NEV):
            step(i0 + bb, bb % NBUF, bb)
        return carry

    lax.fori_loop(0, NBLK, block, 0)
    wait_scatter((NCHUNK_P - 2) % NBUF)
    wait_scatter((NCHUNK_P - 1) % NBUF)

    plsc.subcore_barrier()
    pltpu.sync_copy(acc_sh.at[pl.ds(row0, ROWS_PER_TILE)],
                    out_hbm.at[c, pl.ds(row0, ROWS_PER_TILE)])


def _spmm(support, src, dst, edge_weight):
    mesh = plsc.VectorSubcoreMesh(core_axis_name="c", subcore_axis_name="s")
    k = pl.kernel(
        _spmm_body,
        out_type=jax.ShapeDtypeStruct((NC, N_PAD, D), jnp.float32),
        mesh=mesh,
        scratch_types=(
            [pltpu.VMEM((C, D), jnp.float32)] * NBUF
            + [pltpu.VMEM((C,), jnp.int32)] * NEV
            + [pltpu.VMEM((C,), jnp.int32)] * NEV
            + [pltpu.VMEM((C,), jnp.float32)] * NEV
            + [pltpu.VMEM_SHARED((N_PAD, D), jnp.float32)]
            + [pltpu.SemaphoreType.DMA] * (NBUF + NBUF + 3 * NEV)
        ),
    )
    return k(support, src, dst, edge_weight)


def kernel(inputs, edge_index, edge_weight, weight):
    support = _matmul(inputs, weight)
    partials = _spmm(support, edge_index[1], edge_index[0], edge_weight)
    return _combine(partials)
